# baseline (speedup 1.0000x reference)
import functools

import jax
import jax.numpy as jnp
from jax import lax
from jax.experimental import pallas as pl
from jax.experimental.pallas import tpu as pltpu
from jax.experimental.pallas import tpu_sc as plsc

SEQ = 8192
DIM = 64
NUM_CORES = 2
NUM_SUBCORES = 16
NUM_WORKERS = NUM_CORES * NUM_SUBCORES
BPW = SEQ // NUM_WORKERS
ICHUNK = 128
NCHUNK = BPW // ICHUNK
LANES = 16
VPR = DIM // LANES

_mesh = plsc.VectorSubcoreMesh(
    core_axis_name="c", subcore_axis_name="s", num_cores=NUM_CORES
)


@functools.partial(
    pl.kernel,
    out_type=jax.ShapeDtypeStruct((SEQ, DIM), jnp.float32),
    mesh=_mesh,
    compiler_params=pltpu.CompilerParams(use_tc_tiling_on_sc=False),
    scratch_types=[
        pltpu.VMEM((NCHUNK, ICHUNK), jnp.int32),
        pltpu.VMEM((BPW, DIM), jnp.float32),
        pltpu.VMEM((BPW, DIM), jnp.float32),
        pltpu.SemaphoreType.DMA,
        pltpu.SemaphoreType.DMA,
    ],
)
def _seq_emb_kernel(idx_hbm, word_hbm, pos_hbm, out_hbm,
                    idx_v, rows_v, pos_v, gsem, psem):
    wid = lax.axis_index("s") * NUM_CORES + lax.axis_index("c")
    base = wid * BPW

    pltpu.sync_copy(idx_hbm.at[pl.ds(wid * NCHUNK, NCHUNK)], idx_v)

    gathers = [
        pltpu.async_copy(
            word_hbm.at[idx_v.at[c]],
            rows_v.at[pl.ds(c * ICHUNK, ICHUNK)],
            gsem,
        )
        for c in range(NCHUNK)
    ]
    pcopy = pltpu.async_copy(pos_hbm.at[pl.ds(base, BPW)], pos_v, psem)
    for g in gathers:
        g.wait()
    pcopy.wait()

    def add_body(i, _):
        r = i * 4
        for dr in range(4):
            for j in range(VPR):
                sl = pl.ds(j * LANES, LANES)
                rows_v[r + dr, sl] = rows_v[r + dr, sl] + pos_v[r + dr, sl]
        return _

    lax.fori_loop(0, BPW // 4, add_body, None, unroll=False)

    pltpu.sync_copy(rows_v, out_hbm.at[pl.ds(base, BPW)])


def kernel(x, word_emb, pos_emb):
    idx = x.reshape(NUM_WORKERS * NCHUNK, ICHUNK).astype(jnp.int32)
    return _seq_emb_kernel(idx, word_emb, pos_emb)


# speedup vs baseline: 4.0263x; 4.0263x over previous
import functools

import jax
import jax.numpy as jnp
from jax import lax
from jax.experimental import pallas as pl
from jax.experimental.pallas import tpu as pltpu
from jax.experimental.pallas import tpu_sc as plsc

SEQ = 8192
DIM = 64
NUM_CORES = 2
NUM_SUBCORES = 16
NUM_WORKERS = NUM_CORES * NUM_SUBCORES
BPW = SEQ // NUM_WORKERS
LANES = 16
VPD = DIM // LANES
CHUNK = 8
NCHUNKS = BPW // CHUNK

_mesh = plsc.VectorSubcoreMesh(
    core_axis_name="c", subcore_axis_name="s", num_cores=NUM_CORES
)


@functools.partial(
    pl.kernel,
    out_type=jax.ShapeDtypeStruct((DIM, SEQ), jnp.float32),
    mesh=_mesh,
    compiler_params=pltpu.CompilerParams(needs_layout_passes=False),
    scratch_types=[
        pltpu.VMEM((BPW,), jnp.int32),
        pltpu.SMEM((BPW,), jnp.int32),
        pltpu.VMEM((CHUNK, DIM, 128), jnp.float32),
        pltpu.VMEM((DIM, BPW), jnp.float32),
        pltpu.SemaphoreType.DMA,
        pltpu.SemaphoreType.DMA,
    ],
)
def _seq_emb_kernel(idx_hbm, wT_hbm, posT_hbm, outT_hbm,
                    idx_v, idx_s, tiles_v, acc_v, gsem, psem):
    wid = lax.axis_index("s") * NUM_CORES + lax.axis_index("c")
    base = wid * BPW

    pltpu.sync_copy(idx_hbm.at[pl.ds(base, BPW)], idx_v)

    pcopy = pltpu.async_copy(posT_hbm.at[:, pl.ds(base, BPW)], acc_v, psem)

    d_base = lax.iota(jnp.int32, LANES)

    def do_super(s, _):
        vec = idx_v[pl.ds(s * 2 * CHUNK, 2 * CHUNK)]
        for h in range(2):
            fetches = []
            for ci in range(CHUNK):
                v = vec[h * CHUNK + ci]
                c = pl.multiple_of((v >> 7) << 7, 128)
                fetches.append(pltpu.async_copy(
                    wT_hbm.at[:, pl.ds(c, 128)],
                    tiles_v.at[ci],
                    gsem,
                ))
            for f in fetches:
                f.wait()
            for ci in range(CHUNK):
                v = vec[h * CHUNK + ci]
                i = s * 2 * CHUNK + h * CHUNK + ci
                lane = jnp.full((LANES,), v & 127, jnp.int32)
                i_vec = jnp.full((LANES,), i, jnp.int32)
                for j in range(VPD):
                    d_vec = d_base + j * LANES
                    col = plsc.load_gather(tiles_v.at[ci], [d_vec, lane])
                    pos = plsc.load_gather(acc_v, [d_vec, i_vec])
                    plsc.store_scatter(acc_v, [d_vec, i_vec], col + pos)
        return _

    pcopy.wait()
    lax.fori_loop(0, BPW // (2 * CHUNK), do_super, None, unroll=False)

    pltpu.sync_copy(acc_v, outT_hbm.at[:, pl.ds(base, BPW)])


def kernel(x, word_emb, pos_emb):
    idx = x.reshape(-1).astype(jnp.int32)
    outT = _seq_emb_kernel(idx, word_emb.T, pos_emb.T)
    return outT.T


# speedup vs baseline: 4.1083x; 1.0204x over previous
import functools

import jax
import jax.numpy as jnp
from jax import lax
from jax.experimental import pallas as pl
from jax.experimental.pallas import tpu as pltpu
from jax.experimental.pallas import tpu_sc as plsc

SEQ = 8192
DIM = 64
NUM_CORES = 2
NUM_SUBCORES = 16
NUM_WORKERS = NUM_CORES * NUM_SUBCORES
BPW = SEQ // NUM_WORKERS
LANES = 16
VPD = DIM // LANES
CHUNK = 4
NSLOTS = 2

_mesh = plsc.VectorSubcoreMesh(
    core_axis_name="c", subcore_axis_name="s", num_cores=NUM_CORES
)


@functools.partial(
    pl.kernel,
    out_type=jax.ShapeDtypeStruct((DIM, SEQ), jnp.float32),
    mesh=_mesh,
    compiler_params=pltpu.CompilerParams(needs_layout_passes=False),
    scratch_types=[
        pltpu.VMEM((BPW,), jnp.int32),
        pltpu.VMEM((NSLOTS, CHUNK, DIM, 128), jnp.float32),
        pltpu.VMEM((DIM, BPW), jnp.float32),
        pltpu.SemaphoreType.DMA,
        pltpu.SemaphoreType.DMA,
        pltpu.SemaphoreType.DMA,
    ],
)
def _seq_emb_kernel(idx_hbm, wT_hbm, posT_hbm, outT_hbm,
                    idx_v, tiles_v, acc_v, sem0, sem1, psem):
    wid = lax.axis_index("s") * NUM_CORES + lax.axis_index("c")
    base = wid * BPW
    sems = [sem0, sem1]

    pltpu.sync_copy(idx_hbm.at[pl.ds(base, BPW)], idx_v)

    pcopy = pltpu.async_copy(posT_hbm.at[:, pl.ds(base, BPW)], acc_v, psem)

    d_base = lax.iota(jnp.int32, LANES)

    def fire(vec, q, b):
        fetches = []
        for ci in range(CHUNK):
            v = vec[q * CHUNK + ci]
            c = pl.multiple_of((v >> 7) << 7, 128)
            fetches.append(pltpu.async_copy(
                wT_hbm.at[:, pl.ds(c, 128)],
                tiles_v.at[b, ci],
                sems[b],
            ))
        return fetches

    def extract(vec, s, q, b, fetches):
        for f in fetches:
            f.wait()
        for ci in range(CHUNK):
            v = vec[q * CHUNK + ci]
            i = s * 4 * CHUNK + q * CHUNK + ci
            lane = jnp.full((LANES,), v & 127, jnp.int32)
            i_vec = jnp.full((LANES,), i, jnp.int32)
            for j in range(VPD):
                d_vec = d_base + j * LANES
                col = plsc.load_gather(tiles_v.at[b, ci], [d_vec, lane])
                pos = plsc.load_gather(acc_v, [d_vec, i_vec])
                plsc.store_scatter(acc_v, [d_vec, i_vec], col + pos)

    def do_super(s, _):
        vec = idx_v[pl.ds(s * 4 * CHUNK, 16)]
        f0 = fire(vec, 0, 0)
        f1 = fire(vec, 1, 1)
        extract(vec, s, 0, 0, f0)
        f2 = fire(vec, 2, 0)
        extract(vec, s, 1, 1, f1)
        f3 = fire(vec, 3, 1)
        extract(vec, s, 2, 0, f2)
        extract(vec, s, 3, 1, f3)
        return _

    pcopy.wait()
    lax.fori_loop(0, BPW // (4 * CHUNK), do_super, None, unroll=False)

    pltpu.sync_copy(acc_v, outT_hbm.at[:, pl.ds(base, BPW)])


def kernel(x, word_emb, pos_emb):
    idx = x.reshape(-1).astype(jnp.int32)
    outT = _seq_emb_kernel(idx, word_emb.T, pos_emb.T)
    return outT.T


# speedup vs baseline: 4.2257x; 1.0286x over previous
import functools

import jax
import jax.numpy as jnp
from jax import lax
from jax.experimental import pallas as pl
from jax.experimental.pallas import tpu as pltpu
from jax.experimental.pallas import tpu_sc as plsc

SEQ = 8192
DIM = 64
NUM_CORES = 2
NUM_SUBCORES = 16
NUM_WORKERS = NUM_CORES * NUM_SUBCORES
BPW = SEQ // NUM_WORKERS
LANES = 16
VPD = DIM // LANES
CHUNK = 4
NSLOTS = 2

_mesh = plsc.VectorSubcoreMesh(
    core_axis_name="c", subcore_axis_name="s", num_cores=NUM_CORES
)


@functools.partial(
    pl.kernel,
    out_type=jax.ShapeDtypeStruct((DIM, SEQ), jnp.float32),
    mesh=_mesh,
    compiler_params=pltpu.CompilerParams(needs_layout_passes=False),
    scratch_types=[
        pltpu.VMEM((BPW,), jnp.int32),
        pltpu.VMEM((NSLOTS, CHUNK, DIM, 128), jnp.float32),
        pltpu.VMEM((DIM, BPW), jnp.float32),
        pltpu.SemaphoreType.DMA,
        pltpu.SemaphoreType.DMA,
        pltpu.SemaphoreType.DMA,
    ],
)
def _seq_emb_kernel(idx_hbm, wT_hbm, posT_hbm, outT_hbm,
                    idx_v, tiles_v, acc_v, sem0, sem1, psem):
    wid = lax.axis_index("s") * NUM_CORES + lax.axis_index("c")
    base = wid * BPW
    sems = [sem0, sem1]

    pltpu.sync_copy(idx_hbm.at[pl.ds(base, BPW)], idx_v)

    pcopy = pltpu.async_copy(posT_hbm.at[:, pl.ds(base, BPW)], acc_v, psem)

    d_base = lax.iota(jnp.int32, LANES)

    def fire(vec, q, b):
        fetches = []
        for ci in range(CHUNK):
            v = vec[q * CHUNK + ci]
            c = pl.multiple_of((v >> 7) << 7, 128)
            fetches.append(pltpu.async_copy(
                wT_hbm.at[:, pl.ds(c, 128)],
                tiles_v.at[b, ci],
                sems[b],
            ))
        return fetches

    def extract(vec, s, q, b, fetches):
        for f in fetches:
            f.wait()
        for ci in range(CHUNK):
            v = vec[q * CHUNK + ci]
            i = s * 4 * CHUNK + q * CHUNK + ci
            lane = jnp.full((LANES,), v & 127, jnp.int32)
            i_vec = jnp.full((LANES,), i, jnp.int32)
            for j in range(VPD):
                d_vec = d_base + j * LANES
                col = plsc.load_gather(tiles_v.at[b, ci], [d_vec, lane])
                plsc.addupdate_scatter(acc_v, [d_vec, i_vec], col)

    def do_super(s, _):
        vec = idx_v[pl.ds(s * 4 * CHUNK, 16)]
        f0 = fire(vec, 0, 0)
        f1 = fire(vec, 1, 1)
        extract(vec, s, 0, 0, f0)
        f2 = fire(vec, 2, 0)
        extract(vec, s, 1, 1, f1)
        f3 = fire(vec, 3, 1)
        extract(vec, s, 2, 0, f2)
        extract(vec, s, 3, 1, f3)
        return _

    pcopy.wait()
    lax.fori_loop(0, BPW // (4 * CHUNK), do_super, None, unroll=False)

    pltpu.sync_copy(acc_v, outT_hbm.at[:, pl.ds(base, BPW)])


def kernel(x, word_emb, pos_emb):
    idx = x.reshape(-1).astype(jnp.int32)
    outT = _seq_emb_kernel(idx, word_emb.T, pos_emb.T)
    return outT.T


# speedup vs baseline: 4.6971x; 1.1115x over previous
import functools

import jax
import jax.numpy as jnp
from jax import lax
from jax.experimental import pallas as pl
from jax.experimental.pallas import tpu as pltpu
from jax.experimental.pallas import tpu_sc as plsc

SEQ = 8192
DIM = 64
NUM_CORES = 2
NUM_SUBCORES = 16
NUM_WORKERS = NUM_CORES * NUM_SUBCORES
BPW = SEQ // NUM_WORKERS
LANES = 16
VPD = DIM // LANES
CHUNK = 2
NSLOTS = 4
SUPER = 32
NCH = SUPER // CHUNK

_mesh = plsc.VectorSubcoreMesh(
    core_axis_name="c", subcore_axis_name="s", num_cores=NUM_CORES
)


@functools.partial(
    pl.kernel,
    out_type=jax.ShapeDtypeStruct((DIM, SEQ), jnp.float32),
    mesh=_mesh,
    compiler_params=pltpu.CompilerParams(needs_layout_passes=False),
    scratch_types=[
        pltpu.VMEM((BPW,), jnp.int32),
        pltpu.VMEM((NSLOTS, CHUNK, DIM, 128), jnp.float32),
        pltpu.VMEM((DIM, BPW), jnp.float32),
        pltpu.SemaphoreType.DMA,
        pltpu.SemaphoreType.DMA,
        pltpu.SemaphoreType.DMA,
        pltpu.SemaphoreType.DMA,
        pltpu.SemaphoreType.DMA,
    ],
)
def _seq_emb_kernel(idx_hbm, wT_hbm, posT_hbm, outT_hbm,
                    idx_v, tiles_v, acc_v, sem0, sem1, sem2, sem3, psem):
    wid = lax.axis_index("s") * NUM_CORES + lax.axis_index("c")
    base = wid * BPW
    sems = [sem0, sem1, sem2, sem3]

    pltpu.sync_copy(idx_hbm.at[pl.ds(base, BPW)], idx_v)

    pcopy = pltpu.async_copy(posT_hbm.at[:, pl.ds(base, BPW)], acc_v, psem)

    d_base = lax.iota(jnp.int32, LANES)

    def scal(vecs, k):
        return vecs[k // LANES][k % LANES]

    def fire(vecs, q):
        b = q % NSLOTS
        fetches = []
        for ci in range(CHUNK):
            v = scal(vecs, q * CHUNK + ci)
            c = pl.multiple_of((v >> 7) << 7, 128)
            fetches.append(pltpu.async_copy(
                wT_hbm.at[:, pl.ds(c, 128)],
                tiles_v.at[b, ci],
                sems[b],
            ))
        return fetches

    def extract(vecs, s, q, fetches):
        b = q % NSLOTS
        for f in fetches:
            f.wait()
        for ci in range(CHUNK):
            v = scal(vecs, q * CHUNK + ci)
            i = s * SUPER + q * CHUNK + ci
            lane = jnp.full((LANES,), v & 127, jnp.int32)
            i_vec = jnp.full((LANES,), i, jnp.int32)
            for j in range(VPD):
                d_vec = d_base + j * LANES
                col = plsc.load_gather(tiles_v.at[b, ci], [d_vec, lane])
                plsc.addupdate_scatter(acc_v, [d_vec, i_vec], col)

    def do_super(s, _):
        vecs = [idx_v[pl.ds(s * SUPER + k * LANES, LANES)]
                for k in range(SUPER // LANES)]
        fs = {}
        for q in range(NSLOTS):
            fs[q] = fire(vecs, q)
        for q in range(NCH):
            extract(vecs, s, q, fs.pop(q))
            if q + NSLOTS < NCH:
                fs[q + NSLOTS] = fire(vecs, q + NSLOTS)
        return _

    pcopy.wait()
    lax.fori_loop(0, BPW // SUPER, do_super, None, unroll=False)

    pltpu.sync_copy(acc_v, outT_hbm.at[:, pl.ds(base, BPW)])


def kernel(x, word_emb, pos_emb):
    idx = x.reshape(-1).astype(jnp.int32)
    outT = _seq_emb_kernel(idx, word_emb.T, pos_emb.T)
    return outT.T
